# Initial kernel scaffold; baseline (speedup 1.0000x reference)
import functools

import jax
import jax.numpy as jnp
from jax import lax
from jax.experimental import pallas as pl
from jax.experimental.pallas import tpu as pltpu
from jax.experimental.pallas import tpu_sc as plsc

N = 10000
E = 320000
D = 128
BN_EPS = 1e-5

NC = 2
NS = 16
CHUNK = 128
NCHUNKS = E // CHUNK
CH_PER_CORE = NCHUNKS // NC
CH_PER_TILE = CH_PER_CORE // NS
CH_REM = CH_PER_CORE - CH_PER_TILE * NS
ROWS_PER_TILE = N // NS
ZROWS = 125


def _agg_body(x_hbm, src_hbm, dst_hbm, out0, out1,
              src_v, dst_v, rows_v, zero_v, acc, sem):
    c = lax.axis_index("c")
    s = lax.axis_index("s")

    def zfill(r, carry):
        for j in range(D // 16):
            zero_v[r, pl.ds(j * 16, 16)] = jnp.zeros((16,), jnp.float32)
        return carry
    lax.fori_loop(0, ZROWS, zfill, 0)
    base = s * ROWS_PER_TILE
    for k in range(ROWS_PER_TILE // ZROWS):
        pltpu.sync_copy(zero_v, acc.at[pl.ds(base + k * ZROWS, ZROWS)])
    plsc.subcore_barrier()

    start = c * CH_PER_CORE + s * CH_PER_TILE + jnp.minimum(s, CH_REM)
    count = CH_PER_TILE + jnp.where(s < CH_REM, 1, 0)

    def ebody(i, carry):
        off = (start + i) * CHUNK
        pltpu.sync_copy(src_hbm.at[pl.ds(off, CHUNK)], src_v)
        pltpu.async_copy(x_hbm.at[src_v], rows_v, sem).wait()
        pltpu.sync_copy(dst_hbm.at[pl.ds(off, CHUNK)], dst_v.at[0])
        pltpu.sync_copy(rows_v, acc.at[dst_v.at[0]], add=True)
        return carry
    lax.fori_loop(0, count, ebody, 0)
    plsc.subcore_barrier()

    @pl.when(c == 0)
    def _():
        pltpu.sync_copy(acc.at[pl.ds(base, ROWS_PER_TILE)],
                        out0.at[pl.ds(base, ROWS_PER_TILE)])

    @pl.when(c == 1)
    def _():
        pltpu.sync_copy(acc.at[pl.ds(base, ROWS_PER_TILE)],
                        out1.at[pl.ds(base, ROWS_PER_TILE)])


_agg = pl.kernel(
    _agg_body,
    out_type=(jax.ShapeDtypeStruct((N, D), jnp.float32),
              jax.ShapeDtypeStruct((N, D), jnp.float32)),
    mesh=plsc.VectorSubcoreMesh(core_axis_name="c", subcore_axis_name="s",
                                num_cores=NC, num_subcores=NS),
    scratch_types=(
        pltpu.VMEM((CHUNK,), jnp.int32),
        pltpu.VMEM((1, CHUNK), jnp.int32),
        pltpu.VMEM((CHUNK, D), jnp.float32),
        pltpu.VMEM((ZROWS, D), jnp.float32),
        pltpu.VMEM_SHARED((N, D), jnp.float32),
        pltpu.SemaphoreType.DMA,
    ),
)


BLK = 1000


def _mlp1_body(e_ref, x_ref, p0_ref, p1_ref, w_ref, b_ref, s_ref, t_ref,
               o_ref):
    m = x_ref[...] * e_ref[0, 0] + p0_ref[...] + p1_ref[...]
    z = jnp.dot(m, w_ref[...], preferred_element_type=jnp.float32) + b_ref[...]
    o_ref[...] = jnp.maximum(z, 0.0) * s_ref[...] + t_ref[...]


def _mlp2_body(e_ref, h_ref, p0_ref, p1_ref, w2_ref, b2_ref, w3_ref, b3_ref,
               o_ref):
    m = h_ref[...] * e_ref[0, 0] + p0_ref[...] + p1_ref[...]
    r = jnp.maximum(
        jnp.dot(m, w2_ref[...], preferred_element_type=jnp.float32)
        + b2_ref[...], 0.0)
    o_ref[...] = (jnp.dot(r, w3_ref[...], preferred_element_type=jnp.float32)
                  + b3_ref[...])


_scalar_spec = pl.BlockSpec((1, 1), lambda i: (0, 0))
_vec_spec = pl.BlockSpec((1, D), lambda i: (0, 0))
_mat_spec = pl.BlockSpec((D, D), lambda i: (0, 0))
_blk_spec = pl.BlockSpec((BLK, D), lambda i: (i, 0))

_mlp1 = pl.pallas_call(
    _mlp1_body,
    grid=(N // BLK,),
    in_specs=[_scalar_spec, _blk_spec, _blk_spec, _blk_spec,
              _mat_spec, _vec_spec, _vec_spec, _vec_spec],
    out_specs=_blk_spec,
    out_shape=jax.ShapeDtypeStruct((N, D), jnp.float32),
)

_mlp2 = pl.pallas_call(
    _mlp2_body,
    grid=(N // BLK,),
    in_specs=[_scalar_spec, _blk_spec, _blk_spec, _blk_spec,
              _mat_spec, _vec_spec, _mat_spec, _vec_spec],
    out_specs=_blk_spec,
    out_shape=jax.ShapeDtypeStruct((N, D), jnp.float32),
)


def kernel(x, edge_index, eps1, W1, b1, gm1, bm1, g1, be1,
           eps2, W2, b2, gm2, bm2, g2, be2, W3, b3):
    src = edge_index[0]
    dst = edge_index[1]
    inv = 1.0 / jnp.sqrt(jnp.float32(1.0 + BN_EPS))

    s1a = gm1 * inv
    w1f = W1 * s1a[None, :]
    b1f = (b1 * s1a + bm1)[None, :]
    s1b = (g1 * inv)[None, :]
    t1b = be1[None, :]
    s2a = gm2 * inv
    w2f = W2 * s2a[None, :]
    b2f = (b2 * s2a + bm2)[None, :]
    s2b = g2 * inv
    w3f = s2b[:, None] * W3
    b3f = (b3 + be2 @ W3)[None, :]

    e1 = (1.0 + eps1).reshape(1, 1)
    e2 = (1.0 + eps2).reshape(1, 1)

    p0, p1 = _agg(x, src, dst)
    h = _mlp1(e1, x, p0, p1, w1f, b1f, s1b, t1b)
    q0, q1 = _agg(h, src, dst)
    out = _mlp2(e2, h, q0, q1, w2f, b2f, w3f, b3f)
    return out


# speedup vs baseline: 5.8102x; 5.8102x over previous
import functools

import jax
import jax.numpy as jnp
from jax import lax
from jax.experimental import pallas as pl
from jax.experimental.pallas import tpu as pltpu
from jax.experimental.pallas import tpu_sc as plsc

N = 10000
E = 320000
D = 128
BN_EPS = 1e-5

NC = 2
NS = 16
CHUNK = 128
NCHUNKS = E // CHUNK
CH_PER_CORE = NCHUNKS // NC
CH_PER_TILE = CH_PER_CORE // NS
CH_REM = CH_PER_CORE - CH_PER_TILE * NS
DUMP_ROWS = 632
DUMP_LAST = N - 15 * DUMP_ROWS
ZROWS = 104


def _agg_body(x_hbm, src_hbm, dst_hbm, out0, out1,
              src_v, dst_v, rows_v, zero_v, acc, sem):
    c = lax.axis_index("c")
    s = lax.axis_index("s")

    def zfill(r, carry):
        for j in range(D // 16):
            zero_v[r, pl.ds(j * 16, 16)] = jnp.zeros((16,), jnp.float32)
        return carry
    lax.fori_loop(0, ZROWS, zfill, 0)
    zbase = pl.multiple_of(s * 624, 8)
    for k in range(6):
        pltpu.sync_copy(zero_v,
                        acc.at[pl.ds(pl.multiple_of(zbase + k * ZROWS, 8),
                                     ZROWS)])

    @pl.when(s == 0)
    def _():
        pltpu.sync_copy(zero_v.at[pl.ds(0, 16)], acc.at[pl.ds(N - 16, 16)])
    plsc.subcore_barrier()

    start = c * CH_PER_CORE + s * CH_PER_TILE + jnp.minimum(s, CH_REM)
    count = CH_PER_TILE + jnp.where(s < CH_REM, 1, 0)

    def ebody(i, carry):
        off = (start + i) * CHUNK
        pltpu.sync_copy(src_hbm.at[pl.ds(off, CHUNK)], src_v)
        pltpu.async_copy(x_hbm.at[src_v], rows_v, sem).wait()
        pltpu.sync_copy(dst_hbm.at[pl.ds(off, CHUNK)], dst_v.at[0])
        pltpu.sync_copy(rows_v, acc.at[dst_v.at[0]], add=True)
        return carry
    lax.fori_loop(0, count, ebody, 0)
    plsc.subcore_barrier()

    dbase = pl.multiple_of(s * DUMP_ROWS, 8)
    for core, out in ((0, out0), (1, out1)):
        @pl.when(jnp.logical_and(c == core, s < 15))
        def _(out=out, dbase=dbase):
            pltpu.sync_copy(acc.at[pl.ds(dbase, DUMP_ROWS)],
                            out.at[pl.ds(dbase, DUMP_ROWS)])

        @pl.when(jnp.logical_and(c == core, s == 15))
        def _(out=out, dbase=dbase):
            pltpu.sync_copy(acc.at[pl.ds(dbase, DUMP_LAST)],
                            out.at[pl.ds(dbase, DUMP_LAST)])


_agg = pl.kernel(
    _agg_body,
    out_type=(jax.ShapeDtypeStruct((N, D), jnp.float32),
              jax.ShapeDtypeStruct((N, D), jnp.float32)),
    mesh=plsc.VectorSubcoreMesh(core_axis_name="c", subcore_axis_name="s",
                                num_cores=NC, num_subcores=NS),
    scratch_types=(
        pltpu.VMEM((CHUNK,), jnp.int32),
        pltpu.VMEM((1, CHUNK), jnp.int32),
        pltpu.VMEM((CHUNK, D), jnp.float32),
        pltpu.VMEM((ZROWS, D), jnp.float32),
        pltpu.VMEM_SHARED((N, D), jnp.float32),
        pltpu.SemaphoreType.DMA,
    ),
)


BLK = 1000


def _mlp1_body(e_ref, x_ref, p0_ref, p1_ref, w_ref, b_ref, s_ref, t_ref,
               o_ref):
    m = x_ref[...] * e_ref[0, 0] + p0_ref[...] + p1_ref[...]
    z = jnp.dot(m, w_ref[...], preferred_element_type=jnp.float32) + b_ref[...]
    o_ref[...] = jnp.maximum(z, 0.0) * s_ref[...] + t_ref[...]


def _mlp2_body(e_ref, h_ref, p0_ref, p1_ref, w2_ref, b2_ref, w3_ref, b3_ref,
               o_ref):
    m = h_ref[...] * e_ref[0, 0] + p0_ref[...] + p1_ref[...]
    r = jnp.maximum(
        jnp.dot(m, w2_ref[...], preferred_element_type=jnp.float32)
        + b2_ref[...], 0.0)
    o_ref[...] = (jnp.dot(r, w3_ref[...], preferred_element_type=jnp.float32)
                  + b3_ref[...])


_scalar_spec = pl.BlockSpec((1, 1), lambda i: (0, 0))
_vec_spec = pl.BlockSpec((1, D), lambda i: (0, 0))
_mat_spec = pl.BlockSpec((D, D), lambda i: (0, 0))
_blk_spec = pl.BlockSpec((BLK, D), lambda i: (i, 0))

_mlp1 = pl.pallas_call(
    _mlp1_body,
    grid=(N // BLK,),
    in_specs=[_scalar_spec, _blk_spec, _blk_spec, _blk_spec,
              _mat_spec, _vec_spec, _vec_spec, _vec_spec],
    out_specs=_blk_spec,
    out_shape=jax.ShapeDtypeStruct((N, D), jnp.float32),
)

_mlp2 = pl.pallas_call(
    _mlp2_body,
    grid=(N // BLK,),
    in_specs=[_scalar_spec, _blk_spec, _blk_spec, _blk_spec,
              _mat_spec, _vec_spec, _mat_spec, _vec_spec],
    out_specs=_blk_spec,
    out_shape=jax.ShapeDtypeStruct((N, D), jnp.float32),
)


def kernel(x, edge_index, eps1, W1, b1, gm1, bm1, g1, be1,
           eps2, W2, b2, gm2, bm2, g2, be2, W3, b3):
    src = edge_index[0]
    dst = edge_index[1]
    inv = 1.0 / jnp.sqrt(jnp.float32(1.0 + BN_EPS))

    s1a = gm1 * inv
    w1f = W1 * s1a[None, :]
    b1f = (b1 * s1a + bm1)[None, :]
    s1b = (g1 * inv)[None, :]
    t1b = be1[None, :]
    s2a = gm2 * inv
    w2f = W2 * s2a[None, :]
    b2f = (b2 * s2a + bm2)[None, :]
    s2b = g2 * inv
    w3f = s2b[:, None] * W3
    b3f = (b3 + be2 @ W3)[None, :]

    e1 = (1.0 + eps1).reshape(1, 1)
    e2 = (1.0 + eps2).reshape(1, 1)

    p0, p1 = _agg(x, src, dst)
    h = _mlp1(e1, x, p0, p1, w1f, b1f, s1b, t1b)
    q0, q1 = _agg(h, src, dst)
    out = _mlp2(e2, h, q0, q1, w2f, b2f, w3f, b3f)
    return out


# speedup vs baseline: 11.5490x; 1.9877x over previous
import functools

import jax
import jax.numpy as jnp
from jax import lax
from jax.experimental import pallas as pl
from jax.experimental.pallas import tpu as pltpu
from jax.experimental.pallas import tpu_sc as plsc

N = 10000
E = 320000
D = 128
BN_EPS = 1e-5

NC = 2
NS = 16
NW = NC * NS
CHUNK = 125
CH_PER_TILE = E // (NW * CHUNK)
NBUF = 2
DUMP_ROWS = 632
DUMP_LAST = N - 15 * DUMP_ROWS
ZROWS = 104


def _agg_body(x_hbm, src_hbm, dst_hbm, out0, out1,
              src_v, dst_v, rows0, rows1, acc, sem0, sem1):
    c = lax.axis_index("c")
    s = lax.axis_index("s")
    w = c * NS + s
    rows = (rows0, rows1)
    sems = (sem0, sem1)

    crow = pl.multiple_of(w * CH_PER_TILE, 8)
    pltpu.sync_copy(src_hbm.at[pl.ds(crow, CH_PER_TILE)], src_v)

    def zfill(r, carry):
        for j in range(D // 16):
            rows0[r, pl.ds(j * 16, 16)] = jnp.zeros((16,), jnp.float32)
        return carry
    lax.fori_loop(0, ZROWS, zfill, 0)
    zbase = pl.multiple_of(s * 624, 8)
    for k in range(6):
        pltpu.sync_copy(rows0.at[pl.ds(0, ZROWS)],
                        acc.at[pl.ds(pl.multiple_of(zbase + k * ZROWS, 8),
                                     ZROWS)])

    @pl.when(s == 0)
    def _():
        pltpu.sync_copy(rows0.at[pl.ds(0, 16)], acc.at[pl.ds(N - 16, 16)])
    plsc.subcore_barrier()

    for half in range(2):
        hbase = half * (CH_PER_TILE // 2)
        pltpu.sync_copy(
            dst_hbm.at[pl.ds(pl.multiple_of(crow + hbase, 8),
                             CH_PER_TILE // 2)], dst_v)
        for b in range(NBUF):
            pltpu.async_copy(x_hbm.at[src_v.at[hbase + b]], rows[b], sems[b])

        def ebody(g, carry, half=half, hbase=hbase):
            for b in range(NBUF):
                j = g * NBUF + b
                pltpu.make_async_copy(x_hbm.at[src_v.at[hbase + j]], rows[b],
                                      sems[b]).wait()
                pltpu.sync_copy(rows[b], acc.at[dst_v.at[j]], add=True)

                @pl.when(j + NBUF < CH_PER_TILE // 2)
                def _(j=j, b=b, hbase=hbase):
                    pltpu.async_copy(x_hbm.at[src_v.at[hbase + j + NBUF]],
                                     rows[b], sems[b])
            return carry
        lax.fori_loop(0, CH_PER_TILE // 2 // NBUF, ebody, 0)
    plsc.subcore_barrier()

    dbase = pl.multiple_of(s * DUMP_ROWS, 8)
    for core, out in ((0, out0), (1, out1)):
        @pl.when(jnp.logical_and(c == core, s < 15))
        def _(out=out, dbase=dbase):
            pltpu.sync_copy(acc.at[pl.ds(dbase, DUMP_ROWS)],
                            out.at[pl.ds(dbase, DUMP_ROWS)])

        @pl.when(jnp.logical_and(c == core, s == 15))
        def _(out=out, dbase=dbase):
            pltpu.sync_copy(acc.at[pl.ds(dbase, DUMP_LAST)],
                            out.at[pl.ds(dbase, DUMP_LAST)])


_agg = pl.kernel(
    _agg_body,
    out_type=(jax.ShapeDtypeStruct((N, D), jnp.float32),
              jax.ShapeDtypeStruct((N, D), jnp.float32)),
    mesh=plsc.VectorSubcoreMesh(core_axis_name="c", subcore_axis_name="s",
                                num_cores=NC, num_subcores=NS),
    scratch_types=(
        pltpu.VMEM((CH_PER_TILE, CHUNK), jnp.int32),
        pltpu.VMEM((CH_PER_TILE // 2, CHUNK), jnp.int32),
        pltpu.VMEM((CHUNK, D), jnp.float32),
        pltpu.VMEM((CHUNK, D), jnp.float32),
        pltpu.VMEM_SHARED((N, D), jnp.float32),
        pltpu.SemaphoreType.DMA,
        pltpu.SemaphoreType.DMA,
    ),
)


BLK = 1000


def _mlp1_body(e_ref, x_ref, p0_ref, p1_ref, w_ref, b_ref, s_ref, t_ref,
               o_ref):
    m = x_ref[...] * e_ref[0, 0] + p0_ref[...] + p1_ref[...]
    z = jnp.dot(m, w_ref[...], preferred_element_type=jnp.float32) + b_ref[...]
    o_ref[...] = jnp.maximum(z, 0.0) * s_ref[...] + t_ref[...]


def _mlp2_body(e_ref, h_ref, p0_ref, p1_ref, w2_ref, b2_ref, w3_ref, b3_ref,
               o_ref):
    m = h_ref[...] * e_ref[0, 0] + p0_ref[...] + p1_ref[...]
    r = jnp.maximum(
        jnp.dot(m, w2_ref[...], preferred_element_type=jnp.float32)
        + b2_ref[...], 0.0)
    o_ref[...] = (jnp.dot(r, w3_ref[...], preferred_element_type=jnp.float32)
                  + b3_ref[...])


_scalar_spec = pl.BlockSpec((1, 1), lambda i: (0, 0))
_vec_spec = pl.BlockSpec((1, D), lambda i: (0, 0))
_mat_spec = pl.BlockSpec((D, D), lambda i: (0, 0))
_blk_spec = pl.BlockSpec((BLK, D), lambda i: (i, 0))

_mlp1 = pl.pallas_call(
    _mlp1_body,
    grid=(N // BLK,),
    in_specs=[_scalar_spec, _blk_spec, _blk_spec, _blk_spec,
              _mat_spec, _vec_spec, _vec_spec, _vec_spec],
    out_specs=_blk_spec,
    out_shape=jax.ShapeDtypeStruct((N, D), jnp.float32),
)

_mlp2 = pl.pallas_call(
    _mlp2_body,
    grid=(N // BLK,),
    in_specs=[_scalar_spec, _blk_spec, _blk_spec, _blk_spec,
              _mat_spec, _vec_spec, _mat_spec, _vec_spec],
    out_specs=_blk_spec,
    out_shape=jax.ShapeDtypeStruct((N, D), jnp.float32),
)


def kernel(x, edge_index, eps1, W1, b1, gm1, bm1, g1, be1,
           eps2, W2, b2, gm2, bm2, g2, be2, W3, b3):
    src = edge_index[0].reshape(E // CHUNK, CHUNK)
    dst = edge_index[1].reshape(E // CHUNK, CHUNK)
    inv = 1.0 / jnp.sqrt(jnp.float32(1.0 + BN_EPS))

    s1a = gm1 * inv
    w1f = W1 * s1a[None, :]
    b1f = (b1 * s1a + bm1)[None, :]
    s1b = (g1 * inv)[None, :]
    t1b = be1[None, :]
    s2a = gm2 * inv
    w2f = W2 * s2a[None, :]
    b2f = (b2 * s2a + bm2)[None, :]
    s2b = g2 * inv
    w3f = s2b[:, None] * W3
    b3f = (b3 + be2 @ W3)[None, :]

    e1 = (1.0 + eps1).reshape(1, 1)
    e2 = (1.0 + eps2).reshape(1, 1)

    p0, p1 = _agg(x, src, dst)
    h = _mlp1(e1, x, p0, p1, w1f, b1f, s1b, t1b)
    q0, q1 = _agg(h, src, dst)
    out = _mlp2(e2, h, q0, q1, w2f, b2f, w3f, b3f)
    return out


# speedup vs baseline: 13.0371x; 1.1288x over previous
import functools

import jax
import jax.numpy as jnp
from jax import lax
from jax.experimental import pallas as pl
from jax.experimental.pallas import tpu as pltpu
from jax.experimental.pallas import tpu_sc as plsc

N = 10000
E = 320000
D = 128
BN_EPS = 1e-5

NC = 2
NS = 16
NW = NC * NS
EDGES_PER_TILE = E // NW
CHUNK = 80
CH_PER_TILE = EDGES_PER_TILE // CHUNK
NBUF = 3
DUMP_ROWS = 632
DUMP_LAST = N - 15 * DUMP_ROWS
ZROWS = 48


def _agg_body(x_hbm, pk_hbm, out0, out1,
              pk_v, srcr, dstr, rows0, rows1, rows2, acc, sem0, sem1, sem2):
    c = lax.axis_index("c")
    s = lax.axis_index("s")
    w = c * NS + s
    rows = (rows0, rows1, rows2)
    sems = (sem0, sem1, sem2)

    ebase = pl.multiple_of(w * EDGES_PER_TILE, 8)
    pltpu.sync_copy(pk_hbm.at[pl.ds(ebase, EDGES_PER_TILE)], pk_v)

    def zfill(r, carry):
        for j in range(D // 16):
            rows0[r, pl.ds(j * 16, 16)] = jnp.zeros((16,), jnp.float32)
        return carry
    lax.fori_loop(0, ZROWS, zfill, 0)
    zbase = pl.multiple_of(s * 624, 8)
    for k in range(13):
        pltpu.sync_copy(rows0.at[pl.ds(0, ZROWS)],
                        acc.at[pl.ds(pl.multiple_of(zbase + k * ZROWS, 8),
                                     ZROWS)])

    @pl.when(s == 0)
    def _():
        pltpu.sync_copy(rows0.at[pl.ds(0, 16)], acc.at[pl.ds(N - 16, 16)])

    def unpack(j, b):
        base = j * CHUNK
        for t in range(CHUNK // 16):
            v = pk_v[pl.ds(base + t * 16, 16)]
            srcr[b, pl.ds(t * 16, 16)] = lax.bitwise_and(v, 0xFFFF)
            dstr[b, pl.ds(t * 16, 16)] = lax.shift_right_logical(v, 16)

    for b in range(NBUF):
        unpack(b, b)
        pltpu.async_copy(x_hbm.at[srcr.at[b]], rows[b], sems[b])
    plsc.subcore_barrier()

    def ebody(g, carry):
        for b in range(NBUF):
            j = g * NBUF + b
            pltpu.make_async_copy(x_hbm.at[srcr.at[b]], rows[b],
                                  sems[b]).wait()
            pltpu.sync_copy(rows[b], acc.at[dstr.at[b]], add=True)

            @pl.when(j + NBUF < CH_PER_TILE)
            def _(j=j, b=b):
                unpack(j + NBUF, b)
                pltpu.async_copy(x_hbm.at[srcr.at[b]], rows[b], sems[b])
        return carry
    lax.fori_loop(0, CH_PER_TILE // NBUF, ebody, 0)

    for r in range(CH_PER_TILE % NBUF):
        b = (CH_PER_TILE - (CH_PER_TILE % NBUF) + r) % NBUF
        pltpu.make_async_copy(x_hbm.at[srcr.at[b]], rows[b], sems[b]).wait()
        pltpu.sync_copy(rows[b], acc.at[dstr.at[b]], add=True)
    plsc.subcore_barrier()

    dbase = pl.multiple_of(s * DUMP_ROWS, 8)
    for core, out in ((0, out0), (1, out1)):
        @pl.when(jnp.logical_and(c == core, s < 15))
        def _(out=out, dbase=dbase):
            pltpu.sync_copy(acc.at[pl.ds(dbase, DUMP_ROWS)],
                            out.at[pl.ds(dbase, DUMP_ROWS)])

        @pl.when(jnp.logical_and(c == core, s == 15))
        def _(out=out, dbase=dbase):
            pltpu.sync_copy(acc.at[pl.ds(dbase, DUMP_LAST)],
                            out.at[pl.ds(dbase, DUMP_LAST)])


_agg = pl.kernel(
    _agg_body,
    out_type=(jax.ShapeDtypeStruct((N, D), jnp.float32),
              jax.ShapeDtypeStruct((N, D), jnp.float32)),
    mesh=plsc.VectorSubcoreMesh(core_axis_name="c", subcore_axis_name="s",
                                num_cores=NC, num_subcores=NS),
    scratch_types=(
        pltpu.VMEM((EDGES_PER_TILE,), jnp.int32),
        pltpu.VMEM((NBUF, CHUNK), jnp.int32),
        pltpu.VMEM((NBUF, CHUNK), jnp.int32),
        pltpu.VMEM((CHUNK, D), jnp.float32),
        pltpu.VMEM((CHUNK, D), jnp.float32),
        pltpu.VMEM((CHUNK, D), jnp.float32),
        pltpu.VMEM_SHARED((N, D), jnp.float32),
        pltpu.SemaphoreType.DMA,
        pltpu.SemaphoreType.DMA,
        pltpu.SemaphoreType.DMA,
    ),
)


BLK = 1000


def _mlp1_body(e_ref, x_ref, p0_ref, p1_ref, w_ref, b_ref, s_ref, t_ref,
               o_ref):
    m = x_ref[...] * e_ref[0, 0] + p0_ref[...] + p1_ref[...]
    z = jnp.dot(m, w_ref[...], preferred_element_type=jnp.float32) + b_ref[...]
    o_ref[...] = jnp.maximum(z, 0.0) * s_ref[...] + t_ref[...]


def _mlp2_body(e_ref, h_ref, p0_ref, p1_ref, w2_ref, b2_ref, w3_ref, b3_ref,
               o_ref):
    m = h_ref[...] * e_ref[0, 0] + p0_ref[...] + p1_ref[...]
    r = jnp.maximum(
        jnp.dot(m, w2_ref[...], preferred_element_type=jnp.float32)
        + b2_ref[...], 0.0)
    o_ref[...] = (jnp.dot(r, w3_ref[...], preferred_element_type=jnp.float32)
                  + b3_ref[...])


_scalar_spec = pl.BlockSpec((1, 1), lambda i: (0, 0))
_vec_spec = pl.BlockSpec((1, D), lambda i: (0, 0))
_mat_spec = pl.BlockSpec((D, D), lambda i: (0, 0))
_blk_spec = pl.BlockSpec((BLK, D), lambda i: (i, 0))

_mlp1 = pl.pallas_call(
    _mlp1_body,
    grid=(N // BLK,),
    in_specs=[_scalar_spec, _blk_spec, _blk_spec, _blk_spec,
              _mat_spec, _vec_spec, _vec_spec, _vec_spec],
    out_specs=_blk_spec,
    out_shape=jax.ShapeDtypeStruct((N, D), jnp.float32),
)

_mlp2 = pl.pallas_call(
    _mlp2_body,
    grid=(N // BLK,),
    in_specs=[_scalar_spec, _blk_spec, _blk_spec, _blk_spec,
              _mat_spec, _vec_spec, _mat_spec, _vec_spec],
    out_specs=_blk_spec,
    out_shape=jax.ShapeDtypeStruct((N, D), jnp.float32),
)


def kernel(x, edge_index, eps1, W1, b1, gm1, bm1, g1, be1,
           eps2, W2, b2, gm2, bm2, g2, be2, W3, b3):
    pk = jnp.bitwise_or(edge_index[0],
                        jnp.left_shift(edge_index[1], 16))
    inv = 1.0 / jnp.sqrt(jnp.float32(1.0 + BN_EPS))

    s1a = gm1 * inv
    w1f = W1 * s1a[None, :]
    b1f = (b1 * s1a + bm1)[None, :]
    s1b = (g1 * inv)[None, :]
    t1b = be1[None, :]
    s2a = gm2 * inv
    w2f = W2 * s2a[None, :]
    b2f = (b2 * s2a + bm2)[None, :]
    s2b = g2 * inv
    w3f = s2b[:, None] * W3
    b3f = (b3 + be2 @ W3)[None, :]

    e1 = (1.0 + eps1).reshape(1, 1)
    e2 = (1.0 + eps2).reshape(1, 1)

    p0, p1 = _agg(x, pk)
    h = _mlp1(e1, x, p0, p1, w1f, b1f, s1b, t1b)
    q0, q1 = _agg(h, pk)
    out = _mlp2(e2, h, q0, q1, w2f, b2f, w3f, b3f)
    return out


# speedup vs baseline: 13.1068x; 1.0054x over previous
import functools

import jax
import jax.numpy as jnp
from jax import lax
from jax.experimental import pallas as pl
from jax.experimental.pallas import tpu as pltpu
from jax.experimental.pallas import tpu_sc as plsc

N = 10000
E = 320000
D = 128
BN_EPS = 1e-5

NC = 2
NS = 16
NW = NC * NS
EDGES_PER_TILE = E // NW
CHUNK = 80
CH_PER_TILE = EDGES_PER_TILE // CHUNK
NBUF = 3
DUMP_ROWS = 632
DUMP_LAST = N - 15 * DUMP_ROWS
ZROWS = 48


def _agg_body(x_hbm, pk_hbm, out0, out1,
              pk_v, srcr, dstr, rows0, rows1, rows2, acc, sem0, sem1, sem2):
    c = lax.axis_index("c")
    s = lax.axis_index("s")
    w = c * NS + s
    rows = (rows0, rows1, rows2)
    sems = (sem0, sem1, sem2)

    ebase = pl.multiple_of(w * EDGES_PER_TILE, 8)
    pltpu.sync_copy(pk_hbm.at[pl.ds(ebase, EDGES_PER_TILE)], pk_v)

    def zfill(r, carry):
        for j in range(D // 16):
            rows0[r, pl.ds(j * 16, 16)] = jnp.zeros((16,), jnp.float32)
        return carry
    lax.fori_loop(0, ZROWS, zfill, 0)
    zbase = pl.multiple_of(s * 624, 8)
    for k in range(13):
        pltpu.sync_copy(rows0.at[pl.ds(0, ZROWS)],
                        acc.at[pl.ds(pl.multiple_of(zbase + k * ZROWS, 8),
                                     ZROWS)])

    @pl.when(s == 0)
    def _():
        pltpu.sync_copy(rows0.at[pl.ds(0, 16)], acc.at[pl.ds(N - 16, 16)])

    def unpack(j, b):
        base = j * CHUNK
        for t in range(CHUNK // 16):
            v = pk_v[pl.ds(base + t * 16, 16)]
            srcr[b, pl.ds(t * 16, 16)] = lax.bitwise_and(v, 0xFFFF)
            dstr[b, pl.ds(t * 16, 16)] = lax.shift_right_logical(v, 16)

    for b in range(NBUF):
        unpack(b, b)
        pltpu.async_copy(x_hbm.at[srcr.at[b]], rows[b], sems[b])
    plsc.subcore_barrier()

    def ebody(g, carry):
        for b in range(NBUF):
            j = g * NBUF + b
            pltpu.make_async_copy(x_hbm.at[srcr.at[b]], rows[b],
                                  sems[b]).wait()
            pltpu.sync_copy(rows[b], acc.at[dstr.at[b]], add=True)

            @pl.when(j + NBUF < CH_PER_TILE)
            def _(j=j, b=b):
                unpack(j + NBUF, b)
                pltpu.async_copy(x_hbm.at[srcr.at[b]], rows[b], sems[b])
        return carry
    lax.fori_loop(0, CH_PER_TILE // NBUF, ebody, 0)

    for r in range(CH_PER_TILE % NBUF):
        b = (CH_PER_TILE - (CH_PER_TILE % NBUF) + r) % NBUF
        pltpu.make_async_copy(x_hbm.at[srcr.at[b]], rows[b], sems[b]).wait()
        pltpu.sync_copy(rows[b], acc.at[dstr.at[b]], add=True)
    plsc.subcore_barrier()

    dbase = pl.multiple_of(s * DUMP_ROWS, 8)
    for core, out in ((0, out0), (1, out1)):
        @pl.when(jnp.logical_and(c == core, s < 15))
        def _(out=out, dbase=dbase):
            pltpu.sync_copy(acc.at[pl.ds(dbase, DUMP_ROWS)],
                            out.at[pl.ds(dbase, DUMP_ROWS)])

        @pl.when(jnp.logical_and(c == core, s == 15))
        def _(out=out, dbase=dbase):
            pltpu.sync_copy(acc.at[pl.ds(dbase, DUMP_LAST)],
                            out.at[pl.ds(dbase, DUMP_LAST)])


_agg = pl.kernel(
    _agg_body,
    out_type=(jax.ShapeDtypeStruct((N, D), jnp.float32),
              jax.ShapeDtypeStruct((N, D), jnp.float32)),
    mesh=plsc.VectorSubcoreMesh(core_axis_name="c", subcore_axis_name="s",
                                num_cores=NC, num_subcores=NS),
    scratch_types=(
        pltpu.VMEM((EDGES_PER_TILE,), jnp.int32),
        pltpu.VMEM((NBUF, CHUNK), jnp.int32),
        pltpu.VMEM((NBUF, CHUNK), jnp.int32),
        pltpu.VMEM((CHUNK, D), jnp.float32),
        pltpu.VMEM((CHUNK, D), jnp.float32),
        pltpu.VMEM((CHUNK, D), jnp.float32),
        pltpu.VMEM_SHARED((N, D), jnp.float32),
        pltpu.SemaphoreType.DMA,
        pltpu.SemaphoreType.DMA,
        pltpu.SemaphoreType.DMA,
    ),
)


BLK = 2000


def _mlp1_body(e_ref, x_ref, p0_ref, p1_ref, w_ref, b_ref, s_ref, t_ref,
               o_ref):
    m = x_ref[...] * e_ref[0, 0] + p0_ref[...] + p1_ref[...]
    z = jnp.dot(m, w_ref[...], preferred_element_type=jnp.float32) + b_ref[...]
    o_ref[...] = jnp.maximum(z, 0.0) * s_ref[...] + t_ref[...]


def _mlp2_body(e_ref, h_ref, p0_ref, p1_ref, w2_ref, b2_ref, w3_ref, b3_ref,
               o_ref):
    m = h_ref[...] * e_ref[0, 0] + p0_ref[...] + p1_ref[...]
    r = jnp.maximum(
        jnp.dot(m, w2_ref[...], preferred_element_type=jnp.float32)
        + b2_ref[...], 0.0)
    o_ref[...] = (jnp.dot(r, w3_ref[...], preferred_element_type=jnp.float32)
                  + b3_ref[...])


_scalar_spec = pl.BlockSpec((1, 1), lambda i: (0, 0))
_vec_spec = pl.BlockSpec((1, D), lambda i: (0, 0))
_mat_spec = pl.BlockSpec((D, D), lambda i: (0, 0))
_blk_spec = pl.BlockSpec((BLK, D), lambda i: (i, 0))

_mlp1 = pl.pallas_call(
    _mlp1_body,
    grid=(N // BLK,),
    in_specs=[_scalar_spec, _blk_spec, _blk_spec, _blk_spec,
              _mat_spec, _vec_spec, _vec_spec, _vec_spec],
    out_specs=_blk_spec,
    out_shape=jax.ShapeDtypeStruct((N, D), jnp.float32),
)

_mlp2 = pl.pallas_call(
    _mlp2_body,
    grid=(N // BLK,),
    in_specs=[_scalar_spec, _blk_spec, _blk_spec, _blk_spec,
              _mat_spec, _vec_spec, _mat_spec, _vec_spec],
    out_specs=_blk_spec,
    out_shape=jax.ShapeDtypeStruct((N, D), jnp.float32),
)


def kernel(x, edge_index, eps1, W1, b1, gm1, bm1, g1, be1,
           eps2, W2, b2, gm2, bm2, g2, be2, W3, b3):
    ei3 = edge_index.reshape(2, E // 128, 128)
    pk = jnp.bitwise_or(ei3[0], jnp.left_shift(ei3[1], 16)).reshape(E)
    inv = 1.0 / jnp.sqrt(jnp.float32(1.0 + BN_EPS))

    s1a = gm1 * inv
    w1f = W1 * s1a[None, :]
    b1f = (b1 * s1a + bm1)[None, :]
    s1b = (g1 * inv)[None, :]
    t1b = be1[None, :]
    s2a = gm2 * inv
    w2f = W2 * s2a[None, :]
    b2f = (b2 * s2a + bm2)[None, :]
    s2b = g2 * inv
    w3f = s2b[:, None] * W3
    b3f = (b3 + be2 @ W3)[None, :]

    e1 = (1.0 + eps1).reshape(1, 1)
    e2 = (1.0 + eps2).reshape(1, 1)

    p0, p1 = _agg(x, pk)
    h = _mlp1(e1, x, p0, p1, w1f, b1f, s1b, t1b)
    q0, q1 = _agg(h, pk)
    out = _mlp2(e2, h, q0, q1, w2f, b2f, w3f, b3f)
    return out


# speedup vs baseline: 13.5905x; 1.0369x over previous
import functools

import jax
import jax.numpy as jnp
from jax import lax
from jax.experimental import pallas as pl
from jax.experimental.pallas import tpu as pltpu
from jax.experimental.pallas import tpu_sc as plsc

N = 10000
E = 320000
D = 128
BN_EPS = 1e-5

NC = 2
NS = 16
NW = NC * NS
EPT = 10112
EPT_LAST = E - (NW - 1) * EPT
CHUNK = 64
CH_FULL = EPT // CHUNK
CH_LAST = EPT_LAST // CHUNK
NBUF = 3
DUMP_ROWS = 632
DUMP_LAST = N - 15 * DUMP_ROWS
ZROWS = 48


def _agg_body(x_hbm, ei_hbm, out0, out1,
              src_v, dst_v, dstr, rows0, rows1, rows2, acc,
              sem0, sem1, sem2):
    c = lax.axis_index("c")
    s = lax.axis_index("s")
    w = c * NS + s
    rows = (rows0, rows1, rows2)
    sems = (sem0, sem1, sem2)
    ebase = w * EPT
    nchunks = jnp.where(w == NW - 1, CH_LAST, CH_FULL)

    @pl.when(w < NW - 1)
    def _():
        pltpu.sync_copy(ei_hbm.at[0, pl.ds(ebase, EPT)], src_v)
        pltpu.sync_copy(ei_hbm.at[1, pl.ds(ebase, EPT)], dst_v)

    @pl.when(w == NW - 1)
    def _():
        pltpu.sync_copy(ei_hbm.at[0, pl.ds(ebase, EPT_LAST)],
                        src_v.at[pl.ds(0, EPT_LAST)])
        pltpu.sync_copy(ei_hbm.at[1, pl.ds(ebase, EPT_LAST)],
                        dst_v.at[pl.ds(0, EPT_LAST)])

    def stage_dst(j, b):
        base = j * CHUNK
        for t in range(CHUNK // 16):
            dstr[b, pl.ds(t * 16, 16)] = dst_v[pl.ds(base + t * 16, 16)]

    def fire_gather(j, b):
        pltpu.async_copy(x_hbm.at[src_v.at[pl.ds(j * CHUNK, CHUNK)]],
                         rows[b], sems[b])

    for b in (1, 2):
        stage_dst(b, b)
        fire_gather(b, b)

    def zfill(r, carry):
        for j in range(D // 16):
            rows0[r, pl.ds(j * 16, 16)] = jnp.zeros((16,), jnp.float32)
        return carry
    lax.fori_loop(0, ZROWS, zfill, 0)
    zbase = pl.multiple_of(s * 624, 8)
    for k in range(13):
        pltpu.sync_copy(rows0.at[pl.ds(0, ZROWS)],
                        acc.at[pl.ds(pl.multiple_of(zbase + k * ZROWS, 8),
                                     ZROWS)])

    @pl.when(s == 0)
    def _():
        pltpu.sync_copy(rows0.at[pl.ds(0, 16)], acc.at[pl.ds(N - 16, 16)])

    stage_dst(0, 0)
    fire_gather(0, 0)
    plsc.subcore_barrier()

    def ebody(j, carry):
        for b in range(NBUF):
            @pl.when(lax.rem(j, NBUF) == b)
            def _(b=b):
                pltpu.make_async_copy(
                    x_hbm.at[src_v.at[pl.ds(0, CHUNK)]], rows[b],
                    sems[b]).wait()
                pltpu.sync_copy(rows[b], acc.at[dstr.at[b]], add=True)

                @pl.when(j + NBUF < nchunks)
                def _(b=b):
                    stage_dst(j + NBUF, b)
                    fire_gather(j + NBUF, b)
        return carry
    lax.fori_loop(0, nchunks, ebody, 0)
    plsc.subcore_barrier()

    dbase = pl.multiple_of(s * DUMP_ROWS, 8)
    for core, out in ((0, out0), (1, out1)):
        @pl.when(jnp.logical_and(c == core, s < 15))
        def _(out=out, dbase=dbase):
            pltpu.sync_copy(acc.at[pl.ds(dbase, DUMP_ROWS)],
                            out.at[pl.ds(dbase, DUMP_ROWS)])

        @pl.when(jnp.logical_and(c == core, s == 15))
        def _(out=out, dbase=dbase):
            pltpu.sync_copy(acc.at[pl.ds(dbase, DUMP_LAST)],
                            out.at[pl.ds(dbase, DUMP_LAST)])


_agg = pl.kernel(
    _agg_body,
    out_type=(jax.ShapeDtypeStruct((N, D), jnp.float32),
              jax.ShapeDtypeStruct((N, D), jnp.float32)),
    mesh=plsc.VectorSubcoreMesh(core_axis_name="c", subcore_axis_name="s",
                                num_cores=NC, num_subcores=NS),
    scratch_types=(
        pltpu.VMEM((EPT,), jnp.int32),
        pltpu.VMEM((EPT,), jnp.int32),
        pltpu.VMEM((NBUF, CHUNK), jnp.int32),
        pltpu.VMEM((CHUNK, D), jnp.float32),
        pltpu.VMEM((CHUNK, D), jnp.float32),
        pltpu.VMEM((CHUNK, D), jnp.float32),
        pltpu.VMEM_SHARED((N, D), jnp.float32),
        pltpu.SemaphoreType.DMA,
        pltpu.SemaphoreType.DMA,
        pltpu.SemaphoreType.DMA,
    ),
)


BLK = 2000


def _mlp1_body(e_ref, x_ref, p0_ref, p1_ref, w_ref, b_ref, s_ref, t_ref,
               o_ref):
    m = x_ref[...] * e_ref[0, 0] + p0_ref[...] + p1_ref[...]
    z = jnp.dot(m, w_ref[...], preferred_element_type=jnp.float32) + b_ref[...]
    o_ref[...] = jnp.maximum(z, 0.0) * s_ref[...] + t_ref[...]


def _mlp2_body(e_ref, h_ref, p0_ref, p1_ref, w2_ref, b2_ref, w3_ref, b3_ref,
               o_ref):
    m = h_ref[...] * e_ref[0, 0] + p0_ref[...] + p1_ref[...]
    r = jnp.maximum(
        jnp.dot(m, w2_ref[...], preferred_element_type=jnp.float32)
        + b2_ref[...], 0.0)
    o_ref[...] = (jnp.dot(r, w3_ref[...], preferred_element_type=jnp.float32)
                  + b3_ref[...])


_scalar_spec = pl.BlockSpec((1, 1), lambda i: (0, 0))
_vec_spec = pl.BlockSpec((1, D), lambda i: (0, 0))
_mat_spec = pl.BlockSpec((D, D), lambda i: (0, 0))
_blk_spec = pl.BlockSpec((BLK, D), lambda i: (i, 0))

_mlp1 = pl.pallas_call(
    _mlp1_body,
    grid=(N // BLK,),
    in_specs=[_scalar_spec, _blk_spec, _blk_spec, _blk_spec,
              _mat_spec, _vec_spec, _vec_spec, _vec_spec],
    out_specs=_blk_spec,
    out_shape=jax.ShapeDtypeStruct((N, D), jnp.float32),
)

_mlp2 = pl.pallas_call(
    _mlp2_body,
    grid=(N // BLK,),
    in_specs=[_scalar_spec, _blk_spec, _blk_spec, _blk_spec,
              _mat_spec, _vec_spec, _mat_spec, _vec_spec],
    out_specs=_blk_spec,
    out_shape=jax.ShapeDtypeStruct((N, D), jnp.float32),
)


def kernel(x, edge_index, eps1, W1, b1, gm1, bm1, g1, be1,
           eps2, W2, b2, gm2, bm2, g2, be2, W3, b3):
    inv = 1.0 / jnp.sqrt(jnp.float32(1.0 + BN_EPS))

    s1a = gm1 * inv
    w1f = W1 * s1a[None, :]
    b1f = (b1 * s1a + bm1)[None, :]
    s1b = (g1 * inv)[None, :]
    t1b = be1[None, :]
    s2a = gm2 * inv
    w2f = W2 * s2a[None, :]
    b2f = (b2 * s2a + bm2)[None, :]
    s2b = g2 * inv
    w3f = s2b[:, None] * W3
    b3f = (b3 + be2 @ W3)[None, :]

    e1 = (1.0 + eps1).reshape(1, 1)
    e2 = (1.0 + eps2).reshape(1, 1)

    p0, p1 = _agg(x, edge_index)
    h = _mlp1(e1, x, p0, p1, w1f, b1f, s1b, t1b)
    q0, q1 = _agg(h, edge_index)
    out = _mlp2(e2, h, q0, q1, w2f, b2f, w3f, b3f)
    return out
